# baseline (speedup 1.0000x reference)
import functools

import jax
import jax.numpy as jnp
from jax import lax
from jax.experimental import pallas as pl
from jax.experimental.pallas import tpu as pltpu
from jax.experimental.pallas import tpu_sc as plsc

N_CAND = 1000000
N_SAMP = 200000
NUM_SEG = 1024

NC, NS = 2, 16
NW = NC * NS
C = 31232
NCH = 8
CH = C // NCH
TAIL = N_CAND - NW * C
DEPTH = 2
S = 6272
NSP = NW * S
NBINS = 1040

_MESH = plsc.VectorSubcoreMesh(core_axis_name="c", subcore_axis_name="s")
_PARAMS = pltpu.CompilerParams(needs_layout_passes=False)


def _wid():
    return lax.axis_index("s") * NC + lax.axis_index("c")


def _pass1_body(cand_hbm, u_hbm, tab_hbm, e_hbm, pbins_hbm,
                u_v, e_v, r0_v, r1_v, r2_v, i0_v, i1_v, bins_v,
                sem_u, sem_r0, sem_r1, sem_r2, gs0, gs1, wsem):
    wid = _wid()
    base = wid * C
    rows = [r0_v, r1_v, r2_v]
    rsems = [sem_r0, sem_r1, sem_r2]
    idxs = [i0_v, i1_v]
    gsems = [gs0, gs1]

    cp_u = pltpu.async_copy(u_hbm.at[pl.ds(base, C)], u_v, sem_u)

    def rows_dma(c, n=CH):
        src = pl.ds((base + c * CH) * 5, n * 5)
        return pltpu.async_copy(cand_hbm.at[src], rows[c % 3].at[pl.ds(0, n * 5)],
                                rsems[c % 3])

    r = {c: rows_dma(c) for c in range(3)}

    def zero_bins(i, _):
        bins_v[pl.ds(i * 16, 16)] = jnp.zeros((16,), jnp.float32)
        return _
    lax.fori_loop(0, NBINS // 16, zero_bins, None)

    lanes = lax.iota(jnp.int32, 16)
    lane0 = lanes == 0
    l5 = lanes * 5

    def deint(c, n=CH):
        r[c].wait()
        rv = rows[c % 3]
        iv = idxs[c % 2]

        def st(j, _):
            iv[pl.ds(j * 16, 16)] = plsc.load_gather(rv, [j * 80 + l5 + 1])
            return _
        lax.fori_loop(0, n // 16, st, None)

    def gather(c, n=CH):
        sl = pl.ds(c * CH, n)
        return pltpu.async_copy(tab_hbm.at[idxs[c % 2].at[pl.ds(0, n)]],
                                e_v.at[sl], gsems[c % 2])

    deint(0)
    g = {0: gather(0)}
    deint(1)
    g[1] = gather(1)
    cp_u.wait()

    def make_step(c):
        rv = rows[c % 3]

        def step(j, carry):
            acc, prev = carry
            sl = pl.ds(c * CH + j * 16, 16)
            e16 = jnp.exp(e_v[sl] + u_v[sl])
            e_v[sl] = e16
            seg16 = plsc.load_gather(rv, [j * 80 + l5])
            s0 = seg16[0]
            s15 = seg16[15]
            uniform = jnp.logical_and(s0 == s15, s0 == prev)
            boundary = s0 != s15

            @pl.when(jnp.logical_not(uniform))
            def _flush():
                w = bins_v[pl.ds(prev, 16)]
                bins_v[pl.ds(prev, 16)] = w + jnp.where(lane0, jnp.sum(acc), 0.0)

            @pl.when(boundary)
            def _scatter():
                plsc.addupdate_scatter(bins_v, [seg16], e16)

            acc_n = jnp.where(uniform, acc + e16,
                              jnp.where(boundary, jnp.zeros_like(e16), e16))
            prev_n = jnp.where(uniform, prev, s15)
            return (acc_n, prev_n)
        return step

    acc = jnp.zeros((16,), jnp.float32)
    prev = plsc.load_gather(rows[0], [l5])[0]
    wbs = []
    for c in range(NCH):
        g[c].wait()
        acc, prev = lax.fori_loop(0, CH // 16, make_step(c), (acc, prev))
        csl = pl.ds(c * CH, CH)
        wbs.append(pltpu.async_copy(e_v.at[csl],
                                    e_hbm.at[pl.ds(base + c * CH, CH)], wsem))
        if c + 3 < NCH:
            r[c + 3] = rows_dma(c + 3)
        if c + 2 < NCH:
            deint(c + 2)
            g[c + 2] = gather(c + 2)

    w = bins_v[pl.ds(prev, 16)]
    bins_v[pl.ds(prev, 16)] = w + jnp.where(lane0, jnp.sum(acc), 0.0)

    @pl.when(wid == NW - 1)
    def _tail():
        tbase = NW * C
        tr = pltpu.async_copy(cand_hbm.at[pl.ds(tbase * 5, TAIL * 5)],
                              r0_v.at[pl.ds(0, TAIL * 5)], sem_r0)
        tu = pltpu.async_copy(u_hbm.at[pl.ds(tbase, TAIL)],
                              u_v.at[pl.ds(0, TAIL)], sem_u)
        tr.wait()

        def st(j, _):
            i0_v[pl.ds(j * 16, 16)] = plsc.load_gather(r0_v, [j * 80 + l5 + 1])
            return _
        lax.fori_loop(0, TAIL // 16, st, None)
        pltpu.async_copy(tab_hbm.at[i0_v.at[pl.ds(0, TAIL)]],
                         e_v.at[pl.ds(0, TAIL)], gs0).wait()
        tu.wait()

        def tstep(j, carry):
            acc, prev = carry
            sl = pl.ds(j * 16, 16)
            e16 = jnp.exp(e_v[sl] + u_v[sl])
            e_v[sl] = e16
            seg16 = plsc.load_gather(r0_v, [j * 80 + l5])
            s0 = seg16[0]
            s15 = seg16[15]
            uniform = jnp.logical_and(s0 == s15, s0 == prev)
            boundary = s0 != s15

            @pl.when(jnp.logical_not(uniform))
            def _flush():
                w = bins_v[pl.ds(prev, 16)]
                bins_v[pl.ds(prev, 16)] = w + jnp.where(lane0, jnp.sum(acc), 0.0)

            @pl.when(boundary)
            def _scatter():
                plsc.addupdate_scatter(bins_v, [seg16], e16)

            acc_n = jnp.where(uniform, acc + e16,
                              jnp.where(boundary, jnp.zeros_like(e16), e16))
            prev_n = jnp.where(uniform, prev, s15)
            return (acc_n, prev_n)

        tacc = jnp.zeros((16,), jnp.float32)
        tprev = plsc.load_gather(r0_v, [l5])[0]
        tacc, tprev = lax.fori_loop(0, TAIL // 16, tstep, (tacc, tprev))
        w = bins_v[pl.ds(tprev, 16)]
        bins_v[pl.ds(tprev, 16)] = w + jnp.where(lane0, jnp.sum(tacc), 0.0)
        pltpu.async_copy(e_v.at[pl.ds(0, TAIL)],
                         e_hbm.at[pl.ds(tbase, TAIL)], wsem).wait()

    pltpu.sync_copy(bins_v, pbins_hbm.at[wid])
    for h in wbs:
        h.wait()


def _pass2_body(e_hbm, cand_hbm, pbins_hbm, ca_hbm, y_hbm,
                pb_v, bins_v, ca_v, idx5_v, e_v, seg_v, y_v,
                sem_a, sem_b, sem_c, sem_d):
    wid = _wid()
    base = wid * S

    cp_ca = pltpu.async_copy(ca_hbm.at[pl.ds(base, S)], ca_v, sem_a)
    cp_pb = pltpu.async_copy(pbins_hbm, pb_v, sem_b)
    cp_ca.wait()
    ge = pltpu.async_copy(e_hbm.at[ca_v], e_v, sem_c)

    def mul5(j, _):
        sl = pl.ds(j * 16, 16)
        idx5_v[sl] = ca_v[sl] * 5
        return _
    lax.fori_loop(0, S // 16, mul5, None)
    gs = pltpu.async_copy(cand_hbm.at[idx5_v], seg_v, sem_d)
    cp_pb.wait()

    def red(i, _):
        sl = pl.ds(i * 16, 16)
        acc = pb_v[0, sl]

        def add_row(t, a):
            return a + pb_v[t, sl]
        bins_v[sl] = lax.fori_loop(1, NW, add_row, acc)
        return _
    lax.fori_loop(0, NBINS // 16, red, None)

    ge.wait()
    gs.wait()

    def step(j, _):
        b = j * 16
        denom = plsc.load_gather(bins_v, [seg_v[pl.ds(b, 16)]])
        y = e_v[pl.ds(b, 16)] / denom
        y_v[pl.ds(b, 16)] = (1.0 - y) + y
        return _
    lax.fori_loop(0, S // 16, step, None)

    pltpu.sync_copy(y_v, y_hbm.at[pl.ds(base, S)])


_pass1 = functools.partial(
    pl.kernel,
    out_type=(
        jax.ShapeDtypeStruct((N_CAND,), jnp.float32),
        jax.ShapeDtypeStruct((NW, NBINS), jnp.float32),
    ),
    mesh=_MESH,
    scratch_types=[
        pltpu.VMEM((C,), jnp.float32),
        pltpu.VMEM((C,), jnp.float32),
        pltpu.VMEM((CH * 5,), jnp.int32),
        pltpu.VMEM((CH * 5,), jnp.int32),
        pltpu.VMEM((CH * 5,), jnp.int32),
        pltpu.VMEM((CH,), jnp.int32),
        pltpu.VMEM((CH,), jnp.int32),
        pltpu.VMEM((NBINS,), jnp.float32),
        pltpu.SemaphoreType.DMA,
        pltpu.SemaphoreType.DMA,
        pltpu.SemaphoreType.DMA,
        pltpu.SemaphoreType.DMA,
        pltpu.SemaphoreType.DMA,
        pltpu.SemaphoreType.DMA,
        pltpu.SemaphoreType.DMA,
    ],
    compiler_params=_PARAMS,
)(_pass1_body)

_pass2 = functools.partial(
    pl.kernel,
    out_type=jax.ShapeDtypeStruct((NSP,), jnp.float32),
    mesh=_MESH,
    scratch_types=[
        pltpu.VMEM((NW, NBINS), jnp.float32),
        pltpu.VMEM((NBINS,), jnp.float32),
        pltpu.VMEM((S,), jnp.int32),
        pltpu.VMEM((S,), jnp.int32),
        pltpu.VMEM((S,), jnp.float32),
        pltpu.VMEM((S,), jnp.int32),
        pltpu.VMEM((S,), jnp.float32),
        pltpu.SemaphoreType.DMA,
        pltpu.SemaphoreType.DMA,
        pltpu.SemaphoreType.DMA,
        pltpu.SemaphoreType.DMA,
    ],
    compiler_params=_PARAMS,
)(_pass2_body)


def kernel(candidate_edges, loglog_u, sampled_edges, edges_logits):
    candf = candidate_edges.reshape(-1)
    ca = sampled_edges[:, 5]
    cap = jnp.concatenate([ca, jnp.zeros((NSP - N_SAMP,), jnp.int32)])

    e, pbins = _pass1(candf, loglog_u, edges_logits)
    ypad = _pass2(e, candf, pbins, cap)
    return ypad[:N_SAMP]


# speedup vs baseline: 3.7618x; 3.7618x over previous
import functools

import jax
import jax.numpy as jnp
from jax import lax
from jax.experimental import pallas as pl
from jax.experimental.pallas import tpu as pltpu
from jax.experimental.pallas import tpu_sc as plsc

N_CAND = 1000000
N_SAMP = 200000
NUM_SEG = 1024

NC, NS = 2, 16
NW = NC * NS
C = 31232
NCH = 8
CH = C // NCH
TAIL = N_CAND - NW * C
S = 6272
NSP = NW * S
NBINS = 1040

_MESH = plsc.VectorSubcoreMesh(core_axis_name="c", subcore_axis_name="s")
_PARAMS = pltpu.CompilerParams(needs_layout_passes=False)


def _wid():
    return lax.axis_index("s") * NC + lax.axis_index("c")


def _pass1_body(eid_hbm, u_hbm, eg_hbm, tab_hbm, e_hbm, pbins_hbm,
                e_v, i0_v, i1_v, i2_v, u0_v, u1_v, u2_v, g0_v, g1_v, g2_v,
                bins_v,
                si0, si1, si2, su0, su1, su2, sg0, sg1, sg2, gs0, gs1, gs2,
                wsem):
    wid = _wid()
    base = wid * C
    ivs = [i0_v, i1_v, i2_v]
    uvs = [u0_v, u1_v, u2_v]
    gvs = [g0_v, g1_v, g2_v]
    isems = [si0, si1, si2]
    usems = [su0, su1, su2]
    egsems = [sg0, sg1, sg2]
    gsems = [gs0, gs1, gs2]

    def in_dma(hbm, c, bufs, sems):
        return pltpu.async_copy(hbm.at[pl.ds(base + c * CH, CH)],
                                bufs[c % 3], sems[c % 3])

    def gather(c):
        return pltpu.async_copy(tab_hbm.at[ivs[c % 3]],
                                e_v.at[pl.ds(c * CH, CH)], gsems[c % 3])

    ei = {c: in_dma(eid_hbm, c, ivs, isems) for c in range(3)}
    uu = {c: in_dma(u_hbm, c, uvs, usems) for c in range(2)}
    ee = {c: in_dma(eg_hbm, c, gvs, egsems) for c in range(2)}

    def zero_bins(i, _):
        bins_v[pl.ds(i * 16, 16)] = jnp.zeros((16,), jnp.float32)
        return _
    lax.fori_loop(0, NBINS // 16, zero_bins, None)

    lanes = lax.iota(jnp.int32, 16)
    lane0 = lanes == 0

    ei[0].wait()
    g = {0: gather(0)}
    ei[1].wait()
    g[1] = gather(1)
    ee[0].wait()

    def make_step(uv, gv, off):
        def step(j, carry):
            acc, prev = carry
            sl = pl.ds(off + j * 16, 16)
            jsl = pl.ds(j * 16, 16)
            e16 = jnp.exp(e_v[sl] + uv[jsl])
            e_v[sl] = e16
            seg16 = gv[jsl]
            s0 = seg16[0]
            s15 = seg16[15]
            uniform = jnp.logical_and(s0 == s15, s0 == prev)
            boundary = s0 != s15

            @pl.when(jnp.logical_not(uniform))
            def _flush():
                w = bins_v[pl.ds(prev, 16)]
                bins_v[pl.ds(prev, 16)] = w + jnp.where(lane0, jnp.sum(acc), 0.0)

            @pl.when(boundary)
            def _scatter():
                plsc.addupdate_scatter(bins_v, [seg16], e16)

            acc_n = jnp.where(uniform, acc + e16,
                              jnp.where(boundary, jnp.zeros_like(e16), e16))
            prev_n = jnp.where(uniform, prev, s15)
            return (acc_n, prev_n)
        return step

    acc = jnp.zeros((16,), jnp.float32)
    prev = g0_v[pl.ds(0, 16)][0]
    wbs = []
    for c in range(NCH):
        if c + 2 < NCH:
            ei[c + 2].wait()
            g[c + 2] = gather(c + 2)
            uu[c + 2] = in_dma(u_hbm, c + 2, uvs, usems)
            ee[c + 2] = in_dma(eg_hbm, c + 2, gvs, egsems)
        g[c].wait()
        uu[c].wait()
        if c > 0:
            ee[c].wait()
        acc, prev = lax.fori_loop(0, CH // 16,
                                  make_step(uvs[c % 3], gvs[c % 3], c * CH),
                                  (acc, prev))
        wbs.append(pltpu.async_copy(e_v.at[pl.ds(c * CH, CH)],
                                    e_hbm.at[pl.ds(base + c * CH, CH)], wsem))
        if c + 3 < NCH:
            ei[c + 3] = in_dma(eid_hbm, c + 3, ivs, isems)

    w = bins_v[pl.ds(prev, 16)]
    bins_v[pl.ds(prev, 16)] = w + jnp.where(lane0, jnp.sum(acc), 0.0)

    @pl.when(wid == NW - 1)
    def _tail():
        tbase = NW * C
        ti = pltpu.async_copy(eid_hbm.at[pl.ds(tbase, TAIL)],
                              i0_v.at[pl.ds(0, TAIL)], si0)
        tu = pltpu.async_copy(u_hbm.at[pl.ds(tbase, TAIL)],
                              u0_v.at[pl.ds(0, TAIL)], su0)
        tg = pltpu.async_copy(eg_hbm.at[pl.ds(tbase, TAIL)],
                              g0_v.at[pl.ds(0, TAIL)], sg0)
        ti.wait()
        pltpu.async_copy(tab_hbm.at[i0_v.at[pl.ds(0, TAIL)]],
                         e_v.at[pl.ds(0, TAIL)], gs0).wait()
        tu.wait()
        tg.wait()

        tprev = g0_v[pl.ds(0, 16)][0]
        tacc = jnp.zeros((16,), jnp.float32)
        tacc, tprev = lax.fori_loop(0, TAIL // 16,
                                    make_step(u0_v, g0_v, 0),
                                    (tacc, tprev))
        w = bins_v[pl.ds(tprev, 16)]
        bins_v[pl.ds(tprev, 16)] = w + jnp.where(lane0, jnp.sum(tacc), 0.0)
        pltpu.async_copy(e_v.at[pl.ds(0, TAIL)],
                         e_hbm.at[pl.ds(tbase, TAIL)], wsem).wait()

    pltpu.sync_copy(bins_v, pbins_hbm.at[wid])
    for h in wbs:
        h.wait()


def _pass2_body(e_hbm, eg_hbm, pbins_hbm, ca_hbm, y_hbm,
                pb_v, bins_v, ca_v, e_v, seg_v, y_v,
                sem_a, sem_b, sem_c, sem_d):
    wid = _wid()
    base = wid * S

    cp_ca = pltpu.async_copy(ca_hbm.at[pl.ds(base, S)], ca_v, sem_a)
    cp_pb = pltpu.async_copy(pbins_hbm, pb_v, sem_b)
    cp_ca.wait()
    ge = pltpu.async_copy(e_hbm.at[ca_v], e_v, sem_c)
    gs = pltpu.async_copy(eg_hbm.at[ca_v], seg_v, sem_d)
    cp_pb.wait()

    def red(i, _):
        sl = pl.ds(i * 16, 16)
        acc = pb_v[0, sl]

        def add_row(t, a):
            return a + pb_v[t, sl]
        bins_v[sl] = lax.fori_loop(1, NW, add_row, acc)
        return _
    lax.fori_loop(0, NBINS // 16, red, None)

    ge.wait()
    gs.wait()

    def step(j, _):
        b = j * 16
        denom = plsc.load_gather(bins_v, [seg_v[pl.ds(b, 16)]])
        y = e_v[pl.ds(b, 16)] / denom
        y_v[pl.ds(b, 16)] = (1.0 - y) + y
        return _
    lax.fori_loop(0, S // 16, step, None)

    pltpu.sync_copy(y_v, y_hbm.at[pl.ds(base, S)])


_pass1 = functools.partial(
    pl.kernel,
    out_type=(
        jax.ShapeDtypeStruct((N_CAND,), jnp.float32),
        jax.ShapeDtypeStruct((NW, NBINS), jnp.float32),
    ),
    mesh=_MESH,
    scratch_types=[
        pltpu.VMEM((C,), jnp.float32),
        pltpu.VMEM((CH,), jnp.int32),
        pltpu.VMEM((CH,), jnp.int32),
        pltpu.VMEM((CH,), jnp.int32),
        pltpu.VMEM((CH,), jnp.float32),
        pltpu.VMEM((CH,), jnp.float32),
        pltpu.VMEM((CH,), jnp.float32),
        pltpu.VMEM((CH,), jnp.int32),
        pltpu.VMEM((CH,), jnp.int32),
        pltpu.VMEM((CH,), jnp.int32),
        pltpu.VMEM((NBINS,), jnp.float32),
        pltpu.SemaphoreType.DMA,
        pltpu.SemaphoreType.DMA,
        pltpu.SemaphoreType.DMA,
        pltpu.SemaphoreType.DMA,
        pltpu.SemaphoreType.DMA,
        pltpu.SemaphoreType.DMA,
        pltpu.SemaphoreType.DMA,
        pltpu.SemaphoreType.DMA,
        pltpu.SemaphoreType.DMA,
        pltpu.SemaphoreType.DMA,
        pltpu.SemaphoreType.DMA,
        pltpu.SemaphoreType.DMA,
        pltpu.SemaphoreType.DMA,
    ],
    compiler_params=_PARAMS,
)(_pass1_body)

_pass2 = functools.partial(
    pl.kernel,
    out_type=jax.ShapeDtypeStruct((NSP,), jnp.float32),
    mesh=_MESH,
    scratch_types=[
        pltpu.VMEM((NW, NBINS), jnp.float32),
        pltpu.VMEM((NBINS,), jnp.float32),
        pltpu.VMEM((S,), jnp.int32),
        pltpu.VMEM((S,), jnp.float32),
        pltpu.VMEM((S,), jnp.int32),
        pltpu.VMEM((S,), jnp.float32),
        pltpu.SemaphoreType.DMA,
        pltpu.SemaphoreType.DMA,
        pltpu.SemaphoreType.DMA,
        pltpu.SemaphoreType.DMA,
    ],
    compiler_params=_PARAMS,
)(_pass2_body)


def kernel(candidate_edges, loglog_u, sampled_edges, edges_logits):
    eg = candidate_edges[:, 0]
    eid = candidate_edges[:, 1]
    ca = sampled_edges[:, 5]
    cap = jnp.concatenate([ca, jnp.zeros((NSP - N_SAMP,), jnp.int32)])

    e, pbins = _pass1(eid, loglog_u, eg, edges_logits)
    ypad = _pass2(e, eg, pbins, cap)
    return ypad[:N_SAMP]
